# baseline (speedup 1.0000x reference)
import functools

import jax
import jax.numpy as jnp
from jax import lax
from jax.experimental import pallas as pl
from jax.experimental.pallas import tpu as pltpu
from jax.experimental.pallas import tpu_sc as plsc

H = 4
E = 8
K = 2
D_IN = 1024
FEAT = 1024
N = 8192
HID = 32 * H
BN = 1024

_NEG = -1e30
_L = 16


def _sc_routing(sp_hbm, idx_hbm, probs_hbm, reg_hbm, sp_v, idx_v,
                probs_v, reg_v):
    c = lax.axis_index("c")
    s = lax.axis_index("s")

    @pl.when(jnp.logical_and(c == 0, s == 0))
    def _():
        pltpu.sync_copy(sp_hbm, sp_v)
        lanes = lax.iota(jnp.int32, _L)
        i0s, i1s, deltas = [], [], []
        for h in range(H):
            row = sp_v[h, :]
            m0 = row[0]
            i0 = jnp.int32(0)
            for e in range(1, E):
                ve = row[e]
                take = ve > m0
                m0 = jnp.where(take, ve, m0)
                i0 = jnp.where(take, e, i0)
            m1 = jnp.float32(_NEG)
            i1 = jnp.int32(0)
            for e in range(E):
                ve = row[e]
                take = jnp.logical_and(ve > m1, e != i0)
                m1 = jnp.where(take, ve, m1)
                i1 = jnp.where(take, e, i1)
            i0s.append(i0)
            i1s.append(i1)
            deltas.append(m1 - m0)
        dvec = jnp.zeros((_L,), jnp.float32)
        for h in range(H):
            dvec = jnp.where(lanes == h, deltas[h], dvec)
        ev = jnp.exp(dvec)
        pv = ev / (1.0 + ev)
        p1s = [pv[h] for h in range(H)]
        p0s = [1.0 - p1s[h] for h in range(H)]
        idx_acc = jnp.zeros((_L,), jnp.int32)
        probs_acc = jnp.zeros((_L,), jnp.float32)
        for h in range(H):
            idx_acc = (idx_acc + jnp.where(lanes == 2 * h, i0s[h], 0)
                       + jnp.where(lanes == 2 * h + 1, i1s[h], 0))
            probs_acc = (probs_acc + jnp.where(lanes == 2 * h, p0s[h], 0.0)
                         + jnp.where(lanes == 2 * h + 1, p1s[h], 0.0))
        reg = jnp.float32(0.0)
        for a in range(H):
            gaa = p0s[a] * p0s[a] + p1s[a] * p1s[a]
            d = gaa - 1.0
            reg = reg + d * d
            for b2 in range(a + 1, H):
                gab = jnp.float32(0.0)
                for ia, pa in ((i0s[a], p0s[a]), (i1s[a], p1s[a])):
                    for ib, pb in ((i0s[b2], p0s[b2]), (i1s[b2], p1s[b2])):
                        gab = gab + jnp.where(ia == ib, pa * pb, 0.0)
                reg = reg + 2.0 * gab * gab
        idx_v[...] = idx_acc
        probs_v[...] = probs_acc
        reg_v[...] = jnp.where(lanes == 0, reg, 0.0)
        pltpu.sync_copy(idx_v, idx_hbm)
        pltpu.sync_copy(probs_v, probs_hbm)
        pltpu.sync_copy(reg_v, reg_hbm)


def _mbuild_kernel(idx_ref, probs_ref, W_blk, W1_blk, b_blk, b1_blk,
                   M_ref, beff_ref):
    s = pl.program_id(0)
    h = s // K
    k = s % K
    p = probs_ref[h, k]

    @pl.when(s == 0)
    def _():
        M_ref[...] = jnp.zeros_like(M_ref)
        beff_ref[...] = b1_blk[...]

    Wm = W_blk[0, 0]
    W1m = W1_blk[0]
    bv = b_blk[0]
    M_ref[...] += p * jnp.dot(Wm, W1m, preferred_element_type=jnp.float32)
    beff_ref[...] += p * jnp.dot(bv, W1m, preferred_element_type=jnp.float32)


def _mlp_kernel(x_blk, M_blk, beff_blk, W2_blk, b2_blk, woutT_blk, bout_blk,
                out_ref):
    z1 = jnp.dot(x_blk[...], M_blk[...],
                 preferred_element_type=jnp.float32) + beff_blk[...]
    h1 = jax.nn.softplus(z1)
    z2 = jnp.dot(h1, W2_blk[...],
                 preferred_element_type=jnp.float32) + b2_blk[...]
    h2 = jax.nn.softplus(z2)
    out_ref[...] = (jnp.sum(h2 * woutT_blk[...], axis=1, keepdims=True)
                    + bout_blk[...])


def kernel(x, scaling_params, W, b, W1, b1, W2, b2, Wout, bout):
    f32 = jnp.float32

    sp_pad = jnp.pad(scaling_params, ((0, 0), (0, _L - E)),
                     constant_values=_NEG)

    routing = functools.partial(
        pl.kernel,
        out_type=(
            jax.ShapeDtypeStruct((_L,), jnp.int32),
            jax.ShapeDtypeStruct((_L,), f32),
            jax.ShapeDtypeStruct((_L,), f32),
        ),
        mesh=plsc.VectorSubcoreMesh(core_axis_name="c", subcore_axis_name="s"),
        scratch_types=[
            pltpu.VMEM((H, _L), f32),
            pltpu.VMEM((_L,), jnp.int32),
            pltpu.VMEM((_L,), f32),
            pltpu.VMEM((_L,), f32),
        ],
    )(_sc_routing)
    idx16, probs16, reg16 = routing(sp_pad)
    idx = idx16[:H * K].reshape(H, K)
    probs = probs16[:H * K].reshape(H, K)

    W1r = jnp.transpose(W1.reshape(FEAT, H, HID), (1, 0, 2))
    b_r = b.reshape(H * E, 1, FEAT)
    b1_r = b1.reshape(1, HID)

    grid_spec = pltpu.PrefetchScalarGridSpec(
        num_scalar_prefetch=2,
        grid=(H * K,),
        in_specs=[
            pl.BlockSpec((1, 1, D_IN, FEAT),
                         lambda s, idx_ref, pr_ref: (
                             s // K, idx_ref[s // K, s % K], 0, 0)),
            pl.BlockSpec((1, FEAT, HID),
                         lambda s, idx_ref, pr_ref: (s // K, 0, 0)),
            pl.BlockSpec((1, 1, FEAT),
                         lambda s, idx_ref, pr_ref: (
                             (s // K) * E + idx_ref[s // K, s % K], 0, 0)),
            pl.BlockSpec((1, HID), lambda s, idx_ref, pr_ref: (0, 0)),
        ],
        out_specs=[
            pl.BlockSpec((D_IN, HID), lambda s, idx_ref, pr_ref: (0, 0)),
            pl.BlockSpec((1, HID), lambda s, idx_ref, pr_ref: (0, 0)),
        ],
    )
    M, beff = pl.pallas_call(
        _mbuild_kernel,
        grid_spec=grid_spec,
        out_shape=(
            jax.ShapeDtypeStruct((D_IN, HID), f32),
            jax.ShapeDtypeStruct((1, HID), f32),
        ),
        compiler_params=pltpu.CompilerParams(
            dimension_semantics=("arbitrary",)),
    )(idx, probs, W, W1r, b_r, b1_r)

    out = pl.pallas_call(
        _mlp_kernel,
        grid=(N // BN,),
        in_specs=[
            pl.BlockSpec((BN, D_IN), lambda i: (i, 0)),
            pl.BlockSpec((D_IN, HID), lambda i: (0, 0)),
            pl.BlockSpec((1, HID), lambda i: (0, 0)),
            pl.BlockSpec((HID, HID), lambda i: (0, 0)),
            pl.BlockSpec((1, HID), lambda i: (0, 0)),
            pl.BlockSpec((1, HID), lambda i: (0, 0)),
            pl.BlockSpec((1, 1), lambda i: (0, 0)),
        ],
        out_specs=pl.BlockSpec((BN, 1), lambda i: (i, 0)),
        out_shape=jax.ShapeDtypeStruct((N, 1), f32),
        compiler_params=pltpu.CompilerParams(
            dimension_semantics=("parallel",)),
    )(x, M, beff, W2, b2.reshape(1, HID), Wout.reshape(1, HID),
      bout.reshape(1, 1))

    return out, reg16[0]
